# baseline (speedup 1.0000x reference)
import functools

import jax
import jax.numpy as jnp
from jax import lax
from jax.experimental import pallas as pl
from jax.experimental.pallas import tpu as pltpu
from jax.experimental.pallas import tpu_sc as plsc

NUM_CLASSES = 64
N_CTX = 320000
N_TGT = 32768
D = 128
SCALE = 50.0
EPS = 1e-30

NC = 2
NS = 16
NW = NC * NS

SC_ROWS = 163840
N_TC = N_CTX - SC_ROWS
SC_BLOCKS = SC_ROWS // 2560
CHUNK = 128
RPW = SC_ROWS // NW
NCHUNK = RPW // CHUNK

CTX_BLOCK = 2560
TGT_BLOCK = 2048


_sc_mesh = plsc.VectorSubcoreMesh(core_axis_name="c", subcore_axis_name="s")


@functools.partial(
    pl.kernel,
    mesh=_sc_mesh,
    out_type=jax.ShapeDtypeStruct((NC, NUM_CLASSES, D), jnp.float32),
    scratch_types=[
        pltpu.VMEM((NCHUNK, CHUNK), jnp.int32),
        pltpu.VMEM((CHUNK, D), jnp.float32),
        pltpu.VMEM((CHUNK, D), jnp.float32),
        pltpu.VMEM((NUM_CLASSES, D), jnp.float32),
        pltpu.VMEM_SHARED((NUM_CLASSES, D), jnp.float32),
        pltpu.SemaphoreType.DMA,
        pltpu.SemaphoreType.DMA,
    ],
)
def _sc_segment_sum(ctx_hbm, lab2_hbm, sums_hbm,
                    lab_v, rows0, rows1, zsum_v,
                    acc_sh, sem0, sem1):
    c = lax.axis_index("c")
    s = lax.axis_index("s")
    wid = s * NC + c
    lrow0 = wid * NCHUNK
    pltpu.sync_copy(lab2_hbm.at[pl.ds(lrow0, NCHUNK)], lab_v)

    @pl.when(s == 0)
    def _():
        def zrow(i, _):
            def zcol(j, _):
                zsum_v[i, pl.ds(j * 16, 16)] = jnp.zeros((16,), jnp.float32)
                return 0
            lax.fori_loop(0, D // 16, zcol, 0)
            return 0
        lax.fori_loop(0, NUM_CLASSES, zrow, 0)
        pltpu.sync_copy(zsum_v, acc_sh)

    plsc.subcore_barrier()

    base = wid * RPW
    bufs = (rows0, rows1)
    sems = (sem0, sem1)
    copies = [None, None]
    copies[0] = pltpu.async_copy(
        ctx_hbm.at[pl.ds(base, CHUNK)], rows0, sem0)
    for k in range(NCHUNK):
        cur = k % 2
        nxt = 1 - cur
        if k + 1 < NCHUNK:
            copies[nxt] = pltpu.async_copy(
                ctx_hbm.at[pl.ds(base + (k + 1) * CHUNK, CHUNK)],
                bufs[nxt], sems[nxt])
        copies[cur].wait()
        pltpu.sync_copy(bufs[cur], acc_sh.at[lab_v.at[k]], add=True)

    plsc.subcore_barrier()

    @pl.when(s == 0)
    def _():
        pltpu.sync_copy(acc_sh, sums_hbm.at[c])


def _seg_kernel(lab_ref, ctx_ref, sums_ref, counts_ref):
    @pl.when(pl.program_id(0) == 0)
    def _():
        sums_ref[...] = jnp.zeros_like(sums_ref)
        counts_ref[...] = jnp.zeros_like(counts_ref)

    lab = lab_ref[0, :, :]
    onehot_t = (jax.lax.broadcasted_iota(jnp.int32, (NUM_CLASSES, CTX_BLOCK), 0)
                == lab).astype(jnp.bfloat16)
    counts_ref[...] += jnp.sum(onehot_t.astype(jnp.float32), axis=1,
                               keepdims=True).reshape(1, NUM_CLASSES)

    @pl.when(pl.program_id(0) >= SC_BLOCKS)
    def _():
        ctx = ctx_ref[...]
        sums_ref[...] += jax.lax.dot_general(
            onehot_t, ctx, (((1,), (0,)), ((), ())),
            preferred_element_type=jnp.float32)


def _logit_kernel(sums_ref, counts_ref, scs_ref, tgt_ref,
                  logits_ref, means_ref):
    sums = sums_ref[...] + scs_ref[0] + scs_ref[1]
    counts = counts_ref[...].reshape(NUM_CLASSES, 1)
    means = sums / jnp.maximum(counts, 1.0)
    means_ref[...] = means
    m_norm = jnp.maximum(
        jnp.sqrt(jnp.sum(means * means, axis=1, keepdims=True)), EPS)
    means_scaled = means * (SCALE / m_norm)
    t = tgt_ref[...]
    ones_dc = jnp.ones((D, NUM_CLASSES), dtype=jnp.float32)
    nsq = jax.lax.dot_general(
        t * t, ones_dc, (((1,), (0,)), ((), ())),
        preferred_element_type=jnp.float32)
    inv_t = jax.lax.rsqrt(jnp.maximum(nsq, 1e-37))
    dots = jax.lax.dot_general(
        t, means_scaled, (((1,), (1,)), ((), ())),
        preferred_element_type=jnp.float32)
    logits_ref[...] = dots * inv_t


@jax.jit
def kernel(context_features, context_labels, target_features):
    lab2 = context_labels.reshape(N_CTX // CHUNK, CHUNK)
    sc_sums = _sc_segment_sum(context_features, lab2)

    n_blocks = N_CTX // CTX_BLOCK
    labels3d = context_labels.reshape(N_CTX // CTX_BLOCK, 1, CTX_BLOCK)
    sums, counts = pl.pallas_call(
        _seg_kernel,
        grid=(n_blocks,),
        in_specs=[
            pl.BlockSpec((1, 1, CTX_BLOCK), lambda i: (i, 0, 0)),
            pl.BlockSpec((CTX_BLOCK, D),
                         lambda i: (jnp.maximum(i, SC_BLOCKS), 0)),
        ],
        out_specs=[
            pl.BlockSpec((NUM_CLASSES, D), lambda i: (0, 0)),
            pl.BlockSpec((1, NUM_CLASSES), lambda i: (0, 0)),
        ],
        out_shape=[
            jax.ShapeDtypeStruct((NUM_CLASSES, D), jnp.float32),
            jax.ShapeDtypeStruct((1, NUM_CLASSES), jnp.float32),
        ],
    )(labels3d, context_features)

    t_blocks = N_TGT // TGT_BLOCK
    logits, means = pl.pallas_call(
        _logit_kernel,
        grid=(t_blocks,),
        in_specs=[
            pl.BlockSpec((NUM_CLASSES, D), lambda i: (0, 0)),
            pl.BlockSpec((1, NUM_CLASSES), lambda i: (0, 0)),
            pl.BlockSpec((NC, NUM_CLASSES, D), lambda i: (0, 0, 0)),
            pl.BlockSpec((TGT_BLOCK, D), lambda i: (i, 0)),
        ],
        out_specs=[
            pl.BlockSpec((TGT_BLOCK, NUM_CLASSES), lambda i: (i, 0)),
            pl.BlockSpec((NUM_CLASSES, D), lambda i: (0, 0)),
        ],
        out_shape=[
            jax.ShapeDtypeStruct((N_TGT, NUM_CLASSES), jnp.float32),
            jax.ShapeDtypeStruct((NUM_CLASSES, D), jnp.float32),
        ],
    )(sums, counts, sc_sums, target_features)
    return (logits, means)


# speedup vs baseline: 1.0958x; 1.0958x over previous
import functools

import jax
import jax.numpy as jnp
from jax import lax
from jax.experimental import pallas as pl
from jax.experimental.pallas import tpu as pltpu
from jax.experimental.pallas import tpu_sc as plsc

NUM_CLASSES = 64
N_CTX = 320000
N_TGT = 32768
D = 128
SCALE = 50.0
EPS = 1e-30

NC = 2
NS = 16
NW = NC * NS

CTX_BLOCK = 2560
TGT_BLOCK = 4096

SC_ROWS = 122880
SC_BLOCKS = SC_ROWS // CTX_BLOCK
CHUNK = 128
RPW = SC_ROWS // NW
NCHUNK = RPW // CHUNK


_sc_mesh = plsc.VectorSubcoreMesh(core_axis_name="c", subcore_axis_name="s")


@functools.partial(
    pl.kernel,
    mesh=_sc_mesh,
    out_type=jax.ShapeDtypeStruct((NC, NUM_CLASSES, D), jnp.float32),
    scratch_types=[
        pltpu.VMEM((CHUNK,), jnp.int32),
        pltpu.VMEM((CHUNK,), jnp.int32),
        pltpu.VMEM((CHUNK, D), jnp.float32),
        pltpu.VMEM((CHUNK, D), jnp.float32),
        pltpu.VMEM((NUM_CLASSES, D), jnp.float32),
        pltpu.VMEM_SHARED((NUM_CLASSES, D), jnp.float32),
        pltpu.SemaphoreType.DMA,
        pltpu.SemaphoreType.DMA,
        pltpu.SemaphoreType.DMA,
        pltpu.SemaphoreType.DMA,
    ],
)
def _sc_segment_sum(ctx_hbm, lab_hbm, sums_hbm,
                    idx0, idx1, rows0, rows1, zsum_v,
                    acc_sh, semr0, semr1, semi0, semi1):
    c = lax.axis_index("c")
    s = lax.axis_index("s")
    wid = s * NC + c

    @pl.when(s == 0)
    def _():
        def zrow(i, _):
            def zcol(j, _):
                zsum_v[i, pl.ds(j * 16, 16)] = jnp.zeros((16,), jnp.float32)
                return 0
            lax.fori_loop(0, D // 16, zcol, 0)
            return 0
        lax.fori_loop(0, NUM_CLASSES, zrow, 0)
        pltpu.sync_copy(zsum_v, acc_sh)

    plsc.subcore_barrier()

    base = wid * RPW
    idxs = (idx0, idx1)
    bufs = (rows0, rows1)
    semr = (semr0, semr1)
    semi = (semi0, semi1)
    rcp = [None, None]
    icp = [None, None]
    icp[0] = pltpu.async_copy(lab_hbm.at[pl.ds(base, CHUNK)], idx0, semi0)
    rcp[0] = pltpu.async_copy(ctx_hbm.at[pl.ds(base, CHUNK)], rows0, semr0)
    for k in range(NCHUNK):
        cur = k % 2
        nxt = 1 - cur
        if k + 1 < NCHUNK:
            off = base + (k + 1) * CHUNK
            icp[nxt] = pltpu.async_copy(
                lab_hbm.at[pl.ds(off, CHUNK)], idxs[nxt], semi[nxt])
            rcp[nxt] = pltpu.async_copy(
                ctx_hbm.at[pl.ds(off, CHUNK)], bufs[nxt], semr[nxt])
        icp[cur].wait()
        rcp[cur].wait()
        pltpu.sync_copy(bufs[cur], acc_sh.at[idxs[cur]], add=True)

    plsc.subcore_barrier()

    @pl.when(s == 0)
    def _():
        pltpu.sync_copy(acc_sh, sums_hbm.at[c])


def _seg_kernel(lab_ref, ctx_ref, sums_ref, counts_ref):
    @pl.when(pl.program_id(0) == 0)
    def _():
        sums_ref[...] = jnp.zeros_like(sums_ref)
        counts_ref[...] = jnp.zeros_like(counts_ref)

    lab = lab_ref[pl.ds(pl.program_id(0) * CTX_BLOCK, CTX_BLOCK)
                  ].reshape(1, CTX_BLOCK)
    onehot_t = (jax.lax.broadcasted_iota(jnp.int32, (NUM_CLASSES, CTX_BLOCK), 0)
                == lab).astype(jnp.bfloat16)
    ones_rd = jnp.ones((CTX_BLOCK, D), dtype=jnp.bfloat16)
    counts_ref[...] += jax.lax.dot_general(
        onehot_t, ones_rd, (((1,), (0,)), ((), ())),
        preferred_element_type=jnp.float32)

    @pl.when(pl.program_id(0) >= SC_BLOCKS)
    def _():
        ctx = ctx_ref[...]
        sums_ref[...] += jax.lax.dot_general(
            onehot_t, ctx, (((1,), (0,)), ((), ())),
            preferred_element_type=jnp.float32)


def _logit_kernel(sums_ref, counts_ref, scs_ref, tgt_ref,
                  logits_ref, means_ref):
    sums = sums_ref[...] + scs_ref[0] + scs_ref[1]
    counts = counts_ref[:, 0:1]
    means = sums / jnp.maximum(counts, 1.0)
    means_ref[...] = means
    m_norm = jnp.maximum(
        jnp.sqrt(jnp.sum(means * means, axis=1, keepdims=True)), EPS)
    means_scaled = means * (SCALE / m_norm)
    t = tgt_ref[...]
    ones_dc = jnp.ones((D, NUM_CLASSES), dtype=jnp.float32)
    nsq = jax.lax.dot_general(
        t * t, ones_dc, (((1,), (0,)), ((), ())),
        preferred_element_type=jnp.float32)
    inv_t = jax.lax.rsqrt(jnp.maximum(nsq, 1e-37))
    dots = jax.lax.dot_general(
        t, means_scaled, (((1,), (1,)), ((), ())),
        preferred_element_type=jnp.float32)
    logits_ref[...] = dots * inv_t


@jax.jit
def kernel(context_features, context_labels, target_features):
    sc_sums = _sc_segment_sum(context_features, context_labels)

    n_blocks = N_CTX // CTX_BLOCK
    sums, counts = pl.pallas_call(
        _seg_kernel,
        grid=(n_blocks,),
        in_specs=[
            pl.BlockSpec((N_CTX,), lambda i: (0,)),
            pl.BlockSpec((CTX_BLOCK, D),
                         lambda i: (jnp.maximum(i, SC_BLOCKS), 0)),
        ],
        out_specs=[
            pl.BlockSpec((NUM_CLASSES, D), lambda i: (0, 0)),
            pl.BlockSpec((NUM_CLASSES, D), lambda i: (0, 0)),
        ],
        out_shape=[
            jax.ShapeDtypeStruct((NUM_CLASSES, D), jnp.float32),
            jax.ShapeDtypeStruct((NUM_CLASSES, D), jnp.float32),
        ],
    )(context_labels, context_features)

    t_blocks = N_TGT // TGT_BLOCK
    logits, means = pl.pallas_call(
        _logit_kernel,
        grid=(t_blocks,),
        in_specs=[
            pl.BlockSpec((NUM_CLASSES, D), lambda i: (0, 0)),
            pl.BlockSpec((NUM_CLASSES, D), lambda i: (0, 0)),
            pl.BlockSpec((NC, NUM_CLASSES, D), lambda i: (0, 0, 0)),
            pl.BlockSpec((TGT_BLOCK, D), lambda i: (i, 0)),
        ],
        out_specs=[
            pl.BlockSpec((TGT_BLOCK, NUM_CLASSES), lambda i: (i, 0)),
            pl.BlockSpec((NUM_CLASSES, D), lambda i: (0, 0)),
        ],
        out_shape=[
            jax.ShapeDtypeStruct((N_TGT, NUM_CLASSES), jnp.float32),
            jax.ShapeDtypeStruct((NUM_CLASSES, D), jnp.float32),
        ],
    )(sums, counts, sc_sums, target_features)
    return (logits, means)


# speedup vs baseline: 1.4533x; 1.3263x over previous
import functools

import jax
import jax.numpy as jnp
from jax import lax
from jax.experimental import pallas as pl
from jax.experimental.pallas import tpu as pltpu
from jax.experimental.pallas import tpu_sc as plsc

NUM_CLASSES = 64
N_CTX = 320000
N_TGT = 32768
D = 128
SCALE = 50.0
EPS = 1e-30

NC = 2
NS = 16
NW = NC * NS

CTX_BLOCK = 2560
TGT_BLOCK = 4096

SC_ROWS = 225280
SC_BLOCKS = SC_ROWS // CTX_BLOCK
CHUNK = 128
RPW = SC_ROWS // NW
NCHUNK = RPW // CHUNK


_sc_mesh = plsc.VectorSubcoreMesh(core_axis_name="c", subcore_axis_name="s")


@functools.partial(
    pl.kernel,
    mesh=_sc_mesh,
    out_type=jax.ShapeDtypeStruct((NC, NUM_CLASSES, D), jnp.float32),
    scratch_types=[
        pltpu.VMEM((CHUNK,), jnp.int32),
        pltpu.VMEM((CHUNK,), jnp.int32),
        pltpu.VMEM((CHUNK, D), jnp.float32),
        pltpu.VMEM((CHUNK, D), jnp.float32),
        pltpu.VMEM((NUM_CLASSES, D), jnp.float32),
        pltpu.VMEM_SHARED((NUM_CLASSES, D), jnp.float32),
        pltpu.SemaphoreType.DMA,
        pltpu.SemaphoreType.DMA,
        pltpu.SemaphoreType.DMA,
        pltpu.SemaphoreType.DMA,
    ],
)
def _sc_segment_sum(ctx_hbm, lab_hbm, sums_hbm,
                    idx0, idx1, rows0, rows1, zsum_v,
                    acc_sh, semr0, semr1, semi0, semi1):
    c = lax.axis_index("c")
    s = lax.axis_index("s")
    wid = s * NC + c

    @pl.when(s == 0)
    def _():
        def zrow(i, _):
            def zcol(j, _):
                zsum_v[i, pl.ds(j * 16, 16)] = jnp.zeros((16,), jnp.float32)
                return 0
            lax.fori_loop(0, D // 16, zcol, 0)
            return 0
        lax.fori_loop(0, NUM_CLASSES, zrow, 0)
        pltpu.sync_copy(zsum_v, acc_sh)

    plsc.subcore_barrier()

    base = wid * RPW
    idxs = (idx0, idx1)
    bufs = (rows0, rows1)
    semr = (semr0, semr1)
    semi = (semi0, semi1)
    rcp = [None, None]
    icp = [None, None]
    icp[0] = pltpu.async_copy(lab_hbm.at[pl.ds(base, CHUNK)], idx0, semi0)
    rcp[0] = pltpu.async_copy(ctx_hbm.at[pl.ds(base, CHUNK)], rows0, semr0)
    for k in range(NCHUNK):
        cur = k % 2
        nxt = 1 - cur
        if k + 1 < NCHUNK:
            off = base + (k + 1) * CHUNK
            icp[nxt] = pltpu.async_copy(
                lab_hbm.at[pl.ds(off, CHUNK)], idxs[nxt], semi[nxt])
            rcp[nxt] = pltpu.async_copy(
                ctx_hbm.at[pl.ds(off, CHUNK)], bufs[nxt], semr[nxt])
        icp[cur].wait()
        rcp[cur].wait()
        pltpu.sync_copy(bufs[cur], acc_sh.at[idxs[cur]], add=True)

    plsc.subcore_barrier()

    @pl.when(s == 0)
    def _():
        pltpu.sync_copy(acc_sh, sums_hbm.at[c])


def _seg_kernel(lab_ref, ctx_ref, sums_ref, counts_ref):
    @pl.when(pl.program_id(0) == 0)
    def _():
        sums_ref[...] = jnp.zeros_like(sums_ref)
        counts_ref[...] = jnp.zeros_like(counts_ref)

    lab = lab_ref[pl.ds(pl.program_id(0) * CTX_BLOCK, CTX_BLOCK)
                  ].reshape(1, CTX_BLOCK)
    onehot_t = (jax.lax.broadcasted_iota(jnp.int32, (NUM_CLASSES, CTX_BLOCK), 0)
                == lab).astype(jnp.bfloat16)
    ones_rd = jnp.ones((CTX_BLOCK, D), dtype=jnp.bfloat16)
    counts_ref[...] += jax.lax.dot_general(
        onehot_t, ones_rd, (((1,), (0,)), ((), ())),
        preferred_element_type=jnp.float32)

    @pl.when(pl.program_id(0) >= SC_BLOCKS)
    def _():
        ctx = ctx_ref[...]
        sums_ref[...] += jax.lax.dot_general(
            onehot_t, ctx, (((1,), (0,)), ((), ())),
            preferred_element_type=jnp.float32)


def _logit_kernel(sums_ref, counts_ref, scs_ref, tgt_ref,
                  logits_ref, means_ref):
    sums = sums_ref[...] + scs_ref[0] + scs_ref[1]
    counts = counts_ref[:, 0:1]
    means = sums / jnp.maximum(counts, 1.0)
    means_ref[...] = means
    m_norm = jnp.maximum(
        jnp.sqrt(jnp.sum(means * means, axis=1, keepdims=True)), EPS)
    means_scaled = means * (SCALE / m_norm)
    t = tgt_ref[...]
    ones_cd = jnp.ones((NUM_CLASSES, D), dtype=jnp.float32)
    nsq_t = jax.lax.dot_general(
        ones_cd, t * t, (((1,), (1,)), ((), ())),
        preferred_element_type=jnp.float32)
    inv_t = jax.lax.rsqrt(jnp.maximum(nsq_t, 1e-37))
    dots_t = jax.lax.dot_general(
        means_scaled, t, (((1,), (1,)), ((), ())),
        preferred_element_type=jnp.float32)
    logits_ref[...] = dots_t * inv_t


@jax.jit
def kernel(context_features, context_labels, target_features):
    sc_sums = _sc_segment_sum(context_features, context_labels)

    n_blocks = N_CTX // CTX_BLOCK
    sums, counts = pl.pallas_call(
        _seg_kernel,
        grid=(n_blocks,),
        in_specs=[
            pl.BlockSpec((N_CTX,), lambda i: (0,)),
            pl.BlockSpec((CTX_BLOCK, D),
                         lambda i: (jnp.maximum(i, SC_BLOCKS), 0)),
        ],
        out_specs=[
            pl.BlockSpec((NUM_CLASSES, D), lambda i: (0, 0)),
            pl.BlockSpec((NUM_CLASSES, D), lambda i: (0, 0)),
        ],
        out_shape=[
            jax.ShapeDtypeStruct((NUM_CLASSES, D), jnp.float32),
            jax.ShapeDtypeStruct((NUM_CLASSES, D), jnp.float32),
        ],
    )(context_labels, context_features)

    t_blocks = N_TGT // TGT_BLOCK
    logits, means = pl.pallas_call(
        _logit_kernel,
        grid=(t_blocks,),
        in_specs=[
            pl.BlockSpec((NUM_CLASSES, D), lambda i: (0, 0)),
            pl.BlockSpec((NUM_CLASSES, D), lambda i: (0, 0)),
            pl.BlockSpec((NC, NUM_CLASSES, D), lambda i: (0, 0, 0)),
            pl.BlockSpec((TGT_BLOCK, D), lambda i: (i, 0)),
        ],
        out_specs=[
            pl.BlockSpec((NUM_CLASSES, TGT_BLOCK), lambda i: (0, i)),
            pl.BlockSpec((NUM_CLASSES, D), lambda i: (0, 0)),
        ],
        out_shape=[
            jax.ShapeDtypeStruct((NUM_CLASSES, N_TGT), jnp.float32),
            jax.ShapeDtypeStruct((NUM_CLASSES, D), jnp.float32),
        ],
    )(sums, counts, sc_sums, target_features)
    return (logits.T, means)


# speedup vs baseline: 1.4796x; 1.0181x over previous
import functools

import jax
import jax.numpy as jnp
from jax import lax
from jax.experimental import pallas as pl
from jax.experimental.pallas import tpu as pltpu
from jax.experimental.pallas import tpu_sc as plsc

NUM_CLASSES = 64
N_CTX = 320000
N_TGT = 32768
D = 128
SCALE = 50.0
EPS = 1e-30

NC = 2
NS = 16
NW = NC * NS

CTX_BLOCK = 2560
TGT_BLOCK = 4096

SC_ROWS = 225280
SC_BLOCKS = SC_ROWS // CTX_BLOCK
CHUNK = 128
RPW = SC_ROWS // NW
NCHUNK = RPW // CHUNK


_sc_mesh = plsc.VectorSubcoreMesh(core_axis_name="c", subcore_axis_name="s")


@functools.partial(
    pl.kernel,
    mesh=_sc_mesh,
    out_type=jax.ShapeDtypeStruct((NC, NUM_CLASSES, D), jnp.float32),
    scratch_types=[
        pltpu.VMEM((CHUNK,), jnp.int32),
        pltpu.VMEM((CHUNK,), jnp.int32),
        pltpu.VMEM((CHUNK, D), jnp.float32),
        pltpu.VMEM((CHUNK, D), jnp.float32),
        pltpu.VMEM((NUM_CLASSES, D), jnp.float32),
        pltpu.VMEM_SHARED((NUM_CLASSES, D), jnp.float32),
        pltpu.SemaphoreType.DMA,
        pltpu.SemaphoreType.DMA,
        pltpu.SemaphoreType.DMA,
        pltpu.SemaphoreType.DMA,
    ],
)
def _sc_segment_sum(ctx_hbm, lab_hbm, sums_hbm,
                    idx0, idx1, rows0, rows1, zsum_v,
                    acc_sh, semr0, semr1, semi0, semi1):
    c = lax.axis_index("c")
    s = lax.axis_index("s")
    wid = s * NC + c

    @pl.when(s == 0)
    def _():
        def zrow(i, _):
            def zcol(j, _):
                zsum_v[i, pl.ds(j * 16, 16)] = jnp.zeros((16,), jnp.float32)
                return 0
            lax.fori_loop(0, D // 16, zcol, 0)
            return 0
        lax.fori_loop(0, NUM_CLASSES, zrow, 0)
        pltpu.sync_copy(zsum_v, acc_sh)

    plsc.subcore_barrier()

    base = wid * RPW
    idxs = (idx0, idx1)
    bufs = (rows0, rows1)
    semr = (semr0, semr1)
    semi = (semi0, semi1)

    def _start(chunk, b):
        off = pl.multiple_of(base + chunk * CHUNK, CHUNK)
        pltpu.async_copy(lab_hbm.at[pl.ds(off, CHUNK)], idxs[b], semi[b])
        pltpu.async_copy(ctx_hbm.at[pl.ds(off, CHUNK)], bufs[b], semr[b])

    def _consume(b):
        pltpu.make_async_copy(lab_hbm.at[pl.ds(0, CHUNK)], idxs[b], semi[b]).wait()
        pltpu.make_async_copy(ctx_hbm.at[pl.ds(0, CHUNK)], bufs[b], semr[b]).wait()
        pltpu.sync_copy(bufs[b], acc_sh.at[idxs[b]], add=True)

    _start(0, 0)
    _start(1, 1)

    def ring(i, _):
        k = i * 2
        for b in range(2):
            _consume(b)
            off = pl.multiple_of(base + (k + b + 2) * CHUNK, CHUNK)
            pltpu.async_copy(lab_hbm.at[pl.ds(off, CHUNK)], idxs[b], semi[b])
            pltpu.async_copy(ctx_hbm.at[pl.ds(off, CHUNK)], bufs[b], semr[b])
        return 0
    lax.fori_loop(0, (NCHUNK - 3) // 2, ring, 0)

    _consume(0)
    _start(NCHUNK - 1, 0)
    _consume(1)
    _consume(0)

    plsc.subcore_barrier()

    @pl.when(s == 0)
    def _():
        pltpu.sync_copy(acc_sh, sums_hbm.at[c])


def _seg_kernel(lab_ref, ctx_ref, sums_ref, counts_ref):
    @pl.when(pl.program_id(0) == 0)
    def _():
        sums_ref[...] = jnp.zeros_like(sums_ref)
        counts_ref[...] = jnp.zeros_like(counts_ref)

    lab = lab_ref[pl.ds(pl.program_id(0) * CTX_BLOCK, CTX_BLOCK)
                  ].reshape(1, CTX_BLOCK)
    onehot_t = (jax.lax.broadcasted_iota(jnp.int32, (NUM_CLASSES, CTX_BLOCK), 0)
                == lab).astype(jnp.bfloat16)
    ones_rd = jnp.ones((CTX_BLOCK, D), dtype=jnp.bfloat16)
    counts_ref[...] += jax.lax.dot_general(
        onehot_t, ones_rd, (((1,), (0,)), ((), ())),
        preferred_element_type=jnp.float32)

    @pl.when(pl.program_id(0) >= SC_BLOCKS)
    def _():
        ctx = ctx_ref[...]
        sums_ref[...] += jax.lax.dot_general(
            onehot_t, ctx, (((1,), (0,)), ((), ())),
            preferred_element_type=jnp.float32)


def _logit_kernel(sums_ref, counts_ref, scs_ref, tgt_ref,
                  logits_ref, means_ref):
    sums = sums_ref[...] + scs_ref[0] + scs_ref[1]
    counts = counts_ref[:, 0:1]
    means = sums / jnp.maximum(counts, 1.0)
    means_ref[...] = means
    m_norm = jnp.maximum(
        jnp.sqrt(jnp.sum(means * means, axis=1, keepdims=True)), EPS)
    means_scaled = means * (SCALE / m_norm)
    t = tgt_ref[...]
    ones_cd = jnp.ones((NUM_CLASSES, D), dtype=jnp.float32)
    nsq_t = jax.lax.dot_general(
        ones_cd, t * t, (((1,), (1,)), ((), ())),
        preferred_element_type=jnp.float32)
    inv_t = jax.lax.rsqrt(jnp.maximum(nsq_t, 1e-37))
    dots_t = jax.lax.dot_general(
        means_scaled, t, (((1,), (1,)), ((), ())),
        preferred_element_type=jnp.float32)
    logits_ref[...] = dots_t * inv_t


@jax.jit
def kernel(context_features, context_labels, target_features):
    sc_sums = _sc_segment_sum(context_features, context_labels)

    n_blocks = N_CTX // CTX_BLOCK
    sums, counts = pl.pallas_call(
        _seg_kernel,
        grid=(n_blocks,),
        in_specs=[
            pl.BlockSpec((N_CTX,), lambda i: (0,)),
            pl.BlockSpec((CTX_BLOCK, D),
                         lambda i: (jnp.maximum(i, SC_BLOCKS), 0)),
        ],
        out_specs=[
            pl.BlockSpec((NUM_CLASSES, D), lambda i: (0, 0)),
            pl.BlockSpec((NUM_CLASSES, D), lambda i: (0, 0)),
        ],
        out_shape=[
            jax.ShapeDtypeStruct((NUM_CLASSES, D), jnp.float32),
            jax.ShapeDtypeStruct((NUM_CLASSES, D), jnp.float32),
        ],
    )(context_labels, context_features)

    t_blocks = N_TGT // TGT_BLOCK
    logits, means = pl.pallas_call(
        _logit_kernel,
        grid=(t_blocks,),
        in_specs=[
            pl.BlockSpec((NUM_CLASSES, D), lambda i: (0, 0)),
            pl.BlockSpec((NUM_CLASSES, D), lambda i: (0, 0)),
            pl.BlockSpec((NC, NUM_CLASSES, D), lambda i: (0, 0, 0)),
            pl.BlockSpec((TGT_BLOCK, D), lambda i: (i, 0)),
        ],
        out_specs=[
            pl.BlockSpec((NUM_CLASSES, TGT_BLOCK), lambda i: (0, i)),
            pl.BlockSpec((NUM_CLASSES, D), lambda i: (0, 0)),
        ],
        out_shape=[
            jax.ShapeDtypeStruct((NUM_CLASSES, N_TGT), jnp.float32),
            jax.ShapeDtypeStruct((NUM_CLASSES, D), jnp.float32),
        ],
    )(sums, counts, sc_sums, target_features)
    return (logits.T, means)


# speedup vs baseline: 1.4826x; 1.0020x over previous
import functools

import jax
import jax.numpy as jnp
from jax import lax
from jax.experimental import pallas as pl
from jax.experimental.pallas import tpu as pltpu
from jax.experimental.pallas import tpu_sc as plsc

NUM_CLASSES = 64
N_CTX = 320000
N_TGT = 32768
D = 128
SCALE = 50.0
EPS = 1e-30

NC = 2
NS = 16
NW = NC * NS

CTX_BLOCK = 2560
TGT_BLOCK = 4096

SC_ROWS = 225280
SC_BLOCKS = SC_ROWS // CTX_BLOCK
CHUNK = 128
RPW = SC_ROWS // NW
NCHUNK = RPW // CHUNK


_sc_mesh = plsc.VectorSubcoreMesh(core_axis_name="c", subcore_axis_name="s")


@functools.partial(
    pl.kernel,
    mesh=_sc_mesh,
    out_type=jax.ShapeDtypeStruct((NC, NUM_CLASSES, D), jnp.float32),
    scratch_types=[
        pltpu.VMEM((CHUNK,), jnp.int32),
        pltpu.VMEM((CHUNK,), jnp.int32),
        pltpu.VMEM((CHUNK, D), jnp.float32),
        pltpu.VMEM((CHUNK, D), jnp.float32),
        pltpu.VMEM((NUM_CLASSES, D), jnp.float32),
        pltpu.VMEM_SHARED((NUM_CLASSES, D), jnp.float32),
        pltpu.SemaphoreType.DMA,
        pltpu.SemaphoreType.DMA,
        pltpu.SemaphoreType.DMA,
        pltpu.SemaphoreType.DMA,
    ],
)
def _sc_segment_sum(ctx_hbm, lab_hbm, sums_hbm,
                    idx0, idx1, rows0, rows1, zsum_v,
                    acc_sh, semr0, semr1, semi0, semi1):
    c = lax.axis_index("c")
    s = lax.axis_index("s")
    wid = s * NC + c

    @pl.when(s == 0)
    def _():
        def zsrow(i, _):
            def zscol(j, _):
                zsum_v[i, pl.ds(j * 16, 16)] = jnp.zeros((16,), jnp.float32)
                return 0
            lax.fori_loop(0, D // 16, zscol, 0)
            return 0
        lax.fori_loop(0, NUM_CLASSES, zsrow, 0)
        pltpu.sync_copy(zsum_v, acc_sh)

    plsc.subcore_barrier()

    base = wid * RPW
    idxs = (idx0, idx1)
    bufs = (rows0, rows1)
    semr = (semr0, semr1)
    semi = (semi0, semi1)

    def _start(chunk, b):
        off = pl.multiple_of(base + chunk * CHUNK, CHUNK)
        pltpu.async_copy(lab_hbm.at[pl.ds(off, CHUNK)], idxs[b], semi[b])
        pltpu.async_copy(ctx_hbm.at[pl.ds(off, CHUNK)], bufs[b], semr[b])

    def _consume(b):
        pltpu.make_async_copy(lab_hbm.at[pl.ds(0, CHUNK)], idxs[b], semi[b]).wait()
        pltpu.make_async_copy(ctx_hbm.at[pl.ds(0, CHUNK)], bufs[b], semr[b]).wait()
        pltpu.sync_copy(bufs[b], acc_sh.at[idxs[b]], add=True)

    _start(0, 0)
    _start(1, 1)

    def ring(i, _):
        k = i * 2
        for b in range(2):
            _consume(b)
            off = pl.multiple_of(base + (k + b + 2) * CHUNK, CHUNK)
            pltpu.async_copy(lab_hbm.at[pl.ds(off, CHUNK)], idxs[b], semi[b])
            pltpu.async_copy(ctx_hbm.at[pl.ds(off, CHUNK)], bufs[b], semr[b])
        return 0
    lax.fori_loop(0, (NCHUNK - 3) // 2, ring, 0)

    _consume(0)
    _start(NCHUNK - 1, 0)
    _consume(1)
    _consume(0)

    plsc.subcore_barrier()

    @pl.when(s == 0)
    def _():
        pltpu.sync_copy(acc_sh, sums_hbm.at[c])


def _seg_kernel(lab_ref, ctx_ref, sums_ref, counts_ref):
    @pl.when(pl.program_id(0) == 0)
    def _():
        sums_ref[...] = jnp.zeros_like(sums_ref)
        counts_ref[...] = jnp.zeros_like(counts_ref)

    lab = lab_ref[pl.ds(pl.program_id(0) * CTX_BLOCK, CTX_BLOCK)
                  ].reshape(1, CTX_BLOCK)
    onehot_t = (jax.lax.broadcasted_iota(jnp.int32, (NUM_CLASSES, CTX_BLOCK), 0)
                == lab).astype(jnp.bfloat16)
    ones_rd = jnp.ones((CTX_BLOCK, D), dtype=jnp.bfloat16)
    counts_ref[...] += jax.lax.dot_general(
        onehot_t, ones_rd, (((1,), (0,)), ((), ())),
        preferred_element_type=jnp.float32)

    @pl.when(pl.program_id(0) >= SC_BLOCKS)
    def _():
        ctx = ctx_ref[...]
        sums_ref[...] += jax.lax.dot_general(
            onehot_t, ctx, (((1,), (0,)), ((), ())),
            preferred_element_type=jnp.float32)


def _logit_kernel(sums_ref, counts_ref, scs_ref, tgt_ref,
                  logits_ref, means_ref):
    sums = sums_ref[...] + scs_ref[0] + scs_ref[1]
    counts = counts_ref[:, 0:1]
    means = sums / jnp.maximum(counts, 1.0)
    means_ref[...] = means
    m_norm = jnp.maximum(
        jnp.sqrt(jnp.sum(means * means, axis=1, keepdims=True)), EPS)
    means_scaled = means * (SCALE / m_norm)
    t = tgt_ref[...]
    ones_cd = jnp.ones((NUM_CLASSES, D), dtype=jnp.float32)
    nsq_t = jax.lax.dot_general(
        ones_cd, t * t, (((1,), (1,)), ((), ())),
        preferred_element_type=jnp.float32)
    inv_t = jax.lax.rsqrt(jnp.maximum(nsq_t, 1e-37))
    dots_t = jax.lax.dot_general(
        means_scaled, t, (((1,), (1,)), ((), ())),
        preferred_element_type=jnp.float32)
    logits_ref[...] = dots_t * inv_t


@jax.jit
def kernel(context_features, context_labels, target_features):
    sc_sums = _sc_segment_sum(context_features, context_labels)

    n_blocks = N_CTX // CTX_BLOCK
    sums, counts = pl.pallas_call(
        _seg_kernel,
        grid=(n_blocks,),
        in_specs=[
            pl.BlockSpec((N_CTX,), lambda i: (0,)),
            pl.BlockSpec((CTX_BLOCK, D),
                         lambda i: (jnp.maximum(i, SC_BLOCKS), 0)),
        ],
        out_specs=[
            pl.BlockSpec((NUM_CLASSES, D), lambda i: (0, 0)),
            pl.BlockSpec((NUM_CLASSES, D), lambda i: (0, 0)),
        ],
        out_shape=[
            jax.ShapeDtypeStruct((NUM_CLASSES, D), jnp.float32),
            jax.ShapeDtypeStruct((NUM_CLASSES, D), jnp.float32),
        ],
    )(context_labels, context_features)

    t_blocks = N_TGT // TGT_BLOCK
    logits, means = pl.pallas_call(
        _logit_kernel,
        grid=(t_blocks,),
        in_specs=[
            pl.BlockSpec((NUM_CLASSES, D), lambda i: (0, 0)),
            pl.BlockSpec((NUM_CLASSES, D), lambda i: (0, 0)),
            pl.BlockSpec((NC, NUM_CLASSES, D), lambda i: (0, 0, 0)),
            pl.BlockSpec((TGT_BLOCK, D), lambda i: (i, 0)),
        ],
        out_specs=[
            pl.BlockSpec((NUM_CLASSES, TGT_BLOCK), lambda i: (0, i)),
            pl.BlockSpec((NUM_CLASSES, D), lambda i: (0, 0)),
        ],
        out_shape=[
            jax.ShapeDtypeStruct((NUM_CLASSES, N_TGT), jnp.float32),
            jax.ShapeDtypeStruct((NUM_CLASSES, D), jnp.float32),
        ],
    )(sums, counts, sc_sums, target_features)
    return (logits.T, means)
